# baseline (speedup 1.0000x reference)
import math

import jax
import jax.numpy as jnp
from jax.experimental import pallas as pl

BLK_T = 2048
SPLIT = 16

_TN = (((1,), (1,)), ((), ()))


def _gelu(x):
    return 0.5 * x * (1.0 + jax.lax.erf(x * (1.0 / math.sqrt(2.0))))


def _mmT(a, b):
    return jax.lax.dot_general(a, b, _TN, preferred_element_type=jnp.float32)


def _gate_kernel(s_ref, bk_ref,
                 wq1_ref, bq1_ref, wq2_ref, bq2_ref,
                 wr1b_ref, wr1d_ref, wr1dist_ref, br1_ref,
                 wr2_ref, br2_ref, wv_ref, bv_ref,
                 wrho1q_ref, wrho1z_ref, brho1_ref, wrho2_ref, brho2_ref,
                 ws_ref, bs_ref, wc1_ref, wc2_ref, wt_ref, bt_ref, bc1_ref,
                 alpha_ref, bias1_ref, wtfT_ref, w1b_ref, w2b_ref):
    s = s_ref[...]
    bk = bk_ref[...]
    Bsz = s.shape[0]
    N = bk.shape[0]
    D = wt_ref.shape[0]

    q = _gelu(_mmT(s, wq1_ref[...]) + bq1_ref[...])
    q = _mmT(q, wq2_ref[...]) + bq2_ref[...]
    K = q.shape[-1]

    diff3 = s[:, None, :] - bk[None, :, :]
    dist3 = jnp.sqrt(jnp.sum(diff3 * diff3, axis=-1, keepdims=True))
    diff2 = diff3.reshape(Bsz * N, -1)
    dist2 = dist3.reshape(Bsz * N, 1)
    bank_term = _mmT(bk, wr1b_ref[...]) + br1_ref[...]
    pair = (jnp.broadcast_to(bank_term[None], (Bsz, N, bank_term.shape[-1]))
            .reshape(Bsz * N, -1)
            + _mmT(diff2, wr1d_ref[...])
            + dist2 * wr1dist_ref[...])
    k2 = _mmT(_gelu(pair), wr2_ref[...]) + br2_ref[...]
    k3 = k2.reshape(Bsz, N, K)
    logits = jnp.sum(k3 * q[:, None, :], axis=-1) * (1.0 / math.sqrt(K))
    attn = jax.nn.softmax(logits, axis=-1)
    v = _mmT(bk, wv_ref[...]) + bv_ref[...]
    z = jnp.dot(attn, v)
    rho = _gelu(_mmT(q, wrho1q_ref[...]) + _mmT(z, wrho1z_ref[...])
                + brho1_ref[...])
    alpha = jax.nn.sigmoid(jnp.dot(rho, wrho2_ref[...],
                                   preferred_element_type=jnp.float32)
                           + brho2_ref[...])
    alpha_ref[...] = jnp.broadcast_to(alpha, alpha_ref.shape)

    wc1_t = wc1_ref[:, D:2 * D]
    wc1_s = wc1_ref[:, 2 * D:]
    s_proj = _mmT(s, ws_ref[...]) + bs_ref[...]
    bias1_ref[...] = (_mmT(s_proj, wc1_s) + bc1_ref[...]
                      + _mmT(bt_ref[...], wc1_t))
    wtfT_ref[...] = jnp.dot(wc1_t, wt_ref[...],
                            preferred_element_type=jnp.float32
                            ).astype(jnp.bfloat16)
    w1b_ref[...] = wc1_ref[:, :D].astype(jnp.bfloat16)
    w2b_ref[...] = wc2_ref[...].astype(jnp.bfloat16)


def _mlp_kernel(x_ref, t_ref, alpha_ref, bias1_ref,
                w1_ref, wtfT_ref, w2_ref, bc2_ref, lng_ref, lnb_ref,
                out_ref):
    sub = BLK_T // SPLIT
    D = x_ref.shape[-1]
    bias1 = bias1_ref[0]
    alpha = alpha_ref[0, 0, 0]
    aln = alpha * lng_ref[...]
    alnb = alpha * lnb_ref[...]
    inv_d = 1.0 / D
    for i in range(SPLIT):
        sl = pl.ds(i * sub, sub)
        x = x_ref[0, sl, :]
        h1 = _mmT(x.astype(jnp.bfloat16), w1_ref[...])
        h1 = h1 + _mmT(t_ref[0, sl, :].astype(jnp.bfloat16), wtfT_ref[...])
        h1 = h1 + bias1
        g = _gelu(h1).astype(jnp.bfloat16)
        corr = _mmT(g, w2_ref[...]) + bc2_ref[...]
        mu = jnp.sum(corr, axis=-1, keepdims=True) * inv_d
        m2 = jnp.sum(corr * corr, axis=-1, keepdims=True) * inv_d
        var = m2 - mu * mu
        r = jax.lax.rsqrt(var + 1e-5)
        out_ref[0, sl, :] = x + ((corr - mu) * r) * aln + alnb


def kernel(hidden_states, static_features, time_features, bank,
           Wq1, bq1, Wq2, bq2, Wr1, br1, Wr2, br2, Wv, bv,
           Wrho1, brho1, Wrho2, brho2, Wt, bt, Ws, bs,
           Wc1, bc1, Wc2, bc2, ln_g, ln_b):
    f32 = jnp.float32
    bf16 = jnp.bfloat16
    B, T, D = hidden_states.shape
    NS = static_features.shape[1]
    NT = time_features.shape[2]
    K = Wq2.shape[0]
    DFF = Wc1.shape[0]

    row = lambda b: b.reshape(1, -1)

    alpha8, bias1, wtfT, w1b, w2b = pl.pallas_call(
        _gate_kernel,
        out_shape=[
            jax.ShapeDtypeStruct((B, 128), f32),
            jax.ShapeDtypeStruct((B, DFF), f32),
            jax.ShapeDtypeStruct((DFF, NT), bf16),
            jax.ShapeDtypeStruct((DFF, D), bf16),
            jax.ShapeDtypeStruct((D, DFF), bf16),
        ],
    )(static_features, bank,
      Wq1, row(bq1), Wq2, row(bq2),
      Wr1[:, :NS], Wr1[:, NS:2 * NS], Wr1[:, 2 * NS:].T, row(br1),
      Wr2, row(br2), Wv, row(bv),
      Wrho1[:, :K], Wrho1[:, K:], row(brho1), Wrho2.T, row(brho2),
      Ws, row(bs), Wc1, Wc2, Wt, row(bt), row(bc1))

    n_tblk = T // BLK_T
    out = pl.pallas_call(
        _mlp_kernel,
        grid=(B, n_tblk),
        in_specs=[
            pl.BlockSpec((1, BLK_T, D), lambda b, t: (b, t, 0)),
            pl.BlockSpec((1, BLK_T, NT), lambda b, t: (b, t, 0)),
            pl.BlockSpec((1, 1, 128), lambda b, t: (b, 0, 0)),
            pl.BlockSpec((1, 1, DFF), lambda b, t: (b, 0, 0)),
            pl.BlockSpec((DFF, D), lambda b, t: (0, 0)),
            pl.BlockSpec((DFF, NT), lambda b, t: (0, 0)),
            pl.BlockSpec((D, DFF), lambda b, t: (0, 0)),
            pl.BlockSpec((1, D), lambda b, t: (0, 0)),
            pl.BlockSpec((1, D), lambda b, t: (0, 0)),
            pl.BlockSpec((1, D), lambda b, t: (0, 0)),
        ],
        out_specs=pl.BlockSpec((1, BLK_T, D), lambda b, t: (b, t, 0)),
        out_shape=jax.ShapeDtypeStruct((B, T, D), f32),
    )(hidden_states, time_features,
      alpha8.reshape(B, 1, 128), bias1.reshape(B, 1, DFF),
      w1b, wtfT, w2b,
      row(bc2), row(ln_g), row(ln_b))
    return out


# speedup vs baseline: 2.1080x; 2.1080x over previous
import math

import jax
import jax.numpy as jnp
from jax.experimental import pallas as pl

BLK_T = 2048
SPLIT = 4

_TN = (((1,), (1,)), ((), ()))


def _gelu(x):
    return 0.5 * x * (1.0 + jax.lax.erf(x * (1.0 / math.sqrt(2.0))))


def _mmT(a, b):
    return jax.lax.dot_general(a, b, _TN, preferred_element_type=jnp.float32)


def _gate_kernel(s_ref, bk_ref,
                 wq1_ref, bq1_ref, wq2_ref, bq2_ref,
                 wr1b_ref, wr1d_ref, wr1dist_ref, br1_ref,
                 wr2_ref, br2_ref, wv_ref, bv_ref,
                 wrho1q_ref, wrho1z_ref, brho1_ref, wrho2_ref, brho2_ref,
                 ws_ref, bs_ref, wc1_ref, wc2_ref, wt_ref, bt_ref, bc1_ref,
                 alpha_ref, bias1_ref, wtfT_ref, w1b_ref, w2b_ref):
    s = s_ref[...]
    bk = bk_ref[...]
    Bsz = s.shape[0]
    N = bk.shape[0]
    D = wt_ref.shape[0]

    q = _gelu(_mmT(s, wq1_ref[...]) + bq1_ref[...])
    q = _mmT(q, wq2_ref[...]) + bq2_ref[...]
    K = q.shape[-1]

    diff3 = s[:, None, :] - bk[None, :, :]
    dist3 = jnp.sqrt(jnp.sum(diff3 * diff3, axis=-1, keepdims=True))
    diff2 = diff3.reshape(Bsz * N, -1)
    dist2 = dist3.reshape(Bsz * N, 1)
    bank_term = _mmT(bk, wr1b_ref[...]) + br1_ref[...]
    pair = (jnp.broadcast_to(bank_term[None], (Bsz, N, bank_term.shape[-1]))
            .reshape(Bsz * N, -1)
            + _mmT(diff2, wr1d_ref[...])
            + dist2 * wr1dist_ref[...])
    k2 = _mmT(_gelu(pair), wr2_ref[...]) + br2_ref[...]
    k3 = k2.reshape(Bsz, N, K)
    logits = jnp.sum(k3 * q[:, None, :], axis=-1) * (1.0 / math.sqrt(K))
    attn = jax.nn.softmax(logits, axis=-1)
    v = _mmT(bk, wv_ref[...]) + bv_ref[...]
    z = jnp.dot(attn, v)
    rho = _gelu(_mmT(q, wrho1q_ref[...]) + _mmT(z, wrho1z_ref[...])
                + brho1_ref[...])
    alpha = jax.nn.sigmoid(jnp.dot(rho, wrho2_ref[...],
                                   preferred_element_type=jnp.float32)
                           + brho2_ref[...])
    alpha_ref[...] = jnp.broadcast_to(alpha, alpha_ref.shape)

    wc1_t = wc1_ref[:, D:2 * D]
    wc1_s = wc1_ref[:, 2 * D:]
    s_proj = _mmT(s, ws_ref[...]) + bs_ref[...]
    bias1_ref[...] = (_mmT(s_proj, wc1_s) + bc1_ref[...]
                      + _mmT(bt_ref[...], wc1_t))
    wtfT_ref[...] = jnp.dot(wc1_t, wt_ref[...],
                            preferred_element_type=jnp.float32
                            ).astype(jnp.bfloat16)
    w1b_ref[...] = wc1_ref[:, :D].astype(jnp.bfloat16)
    w2b_ref[...] = wc2_ref[...].astype(jnp.bfloat16)


def _mlp_kernel(x_ref, t_ref, alpha_ref, bias1_ref,
                w1_ref, wtfT_ref, w2_ref, bc2_ref, lng_ref, lnb_ref,
                out_ref):
    sub = BLK_T // SPLIT
    D = x_ref.shape[-1]
    bias1 = bias1_ref[0]
    alpha = alpha_ref[0, 0, 0]
    aln = alpha * lng_ref[...]
    alnb = alpha * lnb_ref[...]
    inv_d = 1.0 / D
    for i in range(SPLIT):
        sl = pl.ds(i * sub, sub)
        x = x_ref[0, sl, :]
        h1 = _mmT(x.astype(jnp.bfloat16), w1_ref[...])
        h1 = h1 + _mmT(t_ref[0, sl, :].astype(jnp.bfloat16), wtfT_ref[...])
        h1 = h1 + bias1
        g = _gelu(h1).astype(jnp.bfloat16)
        corr = _mmT(g, w2_ref[...]) + bc2_ref[...]
        mu = jnp.sum(corr, axis=-1, keepdims=True) * inv_d
        m2 = jnp.sum(corr * corr, axis=-1, keepdims=True) * inv_d
        var = m2 - mu * mu
        r = jax.lax.rsqrt(var + 1e-5)
        out_ref[0, sl, :] = x + ((corr - mu) * r) * aln + alnb


def kernel(hidden_states, static_features, time_features, bank,
           Wq1, bq1, Wq2, bq2, Wr1, br1, Wr2, br2, Wv, bv,
           Wrho1, brho1, Wrho2, brho2, Wt, bt, Ws, bs,
           Wc1, bc1, Wc2, bc2, ln_g, ln_b):
    f32 = jnp.float32
    bf16 = jnp.bfloat16
    B, T, D = hidden_states.shape
    NS = static_features.shape[1]
    NT = time_features.shape[2]
    K = Wq2.shape[0]
    DFF = Wc1.shape[0]

    row = lambda b: b.reshape(1, -1)

    alpha8, bias1, wtfT, w1b, w2b = pl.pallas_call(
        _gate_kernel,
        out_shape=[
            jax.ShapeDtypeStruct((B, 128), f32),
            jax.ShapeDtypeStruct((B, DFF), f32),
            jax.ShapeDtypeStruct((DFF, NT), bf16),
            jax.ShapeDtypeStruct((DFF, D), bf16),
            jax.ShapeDtypeStruct((D, DFF), bf16),
        ],
    )(static_features, bank,
      Wq1, row(bq1), Wq2, row(bq2),
      Wr1[:, :NS], Wr1[:, NS:2 * NS], Wr1[:, 2 * NS:].T, row(br1),
      Wr2, row(br2), Wv, row(bv),
      Wrho1[:, :K], Wrho1[:, K:], row(brho1), Wrho2.T, row(brho2),
      Ws, row(bs), Wc1, Wc2, Wt, row(bt), row(bc1))

    n_tblk = T // BLK_T
    out = pl.pallas_call(
        _mlp_kernel,
        grid=(B, n_tblk),
        in_specs=[
            pl.BlockSpec((1, BLK_T, D), lambda b, t: (b, t, 0)),
            pl.BlockSpec((1, BLK_T, NT), lambda b, t: (b, t, 0)),
            pl.BlockSpec((1, 1, 128), lambda b, t: (b, 0, 0)),
            pl.BlockSpec((1, 1, DFF), lambda b, t: (b, 0, 0)),
            pl.BlockSpec((DFF, D), lambda b, t: (0, 0)),
            pl.BlockSpec((DFF, NT), lambda b, t: (0, 0)),
            pl.BlockSpec((D, DFF), lambda b, t: (0, 0)),
            pl.BlockSpec((1, D), lambda b, t: (0, 0)),
            pl.BlockSpec((1, D), lambda b, t: (0, 0)),
            pl.BlockSpec((1, D), lambda b, t: (0, 0)),
        ],
        out_specs=pl.BlockSpec((1, BLK_T, D), lambda b, t: (b, t, 0)),
        out_shape=jax.ShapeDtypeStruct((B, T, D), f32),
    )(hidden_states, time_features,
      alpha8.reshape(B, 1, 128), bias1.reshape(B, 1, DFF),
      w1b, wtfT, w2b,
      row(bc2), row(ln_g), row(ln_b))
    return out


# speedup vs baseline: 2.1158x; 1.0037x over previous
import math

import jax
import jax.numpy as jnp
from jax.experimental import pallas as pl

BLK_T = 2048
SPLIT = 4
GATE_G = 4

_TN = (((1,), (1,)), ((), ()))


def _gelu(x):
    return 0.5 * x * (1.0 + jax.lax.erf(x * (1.0 / math.sqrt(2.0))))


def _mmT(a, b):
    return jax.lax.dot_general(a, b, _TN, preferred_element_type=jnp.float32)


def _gate_kernel(s_ref, bk_ref,
                 wq1_ref, bq1_ref, wq2_ref, bq2_ref,
                 wr1b_ref, wr1d_ref, wr1dist_ref, br1_ref,
                 wr2_ref, br2_ref, wv_ref, bv_ref,
                 wrho1q_ref, wrho1z_ref, brho1_ref, wrho2_ref, brho2_ref,
                 ws_ref, bs_ref, wc1_ref, wc2_ref, wt_ref, bt_ref, bc1_ref,
                 alpha_ref, bias1_ref, wtfT_ref, w1b_ref, w2b_ref):
    D = wt_ref.shape[0]
    g = pl.program_id(0)

    @pl.when(g == 0)
    def _relational():
        s = s_ref[...]
        bk = bk_ref[...]
        Bsz = s.shape[0]
        N = bk.shape[0]

        q = _gelu(_mmT(s, wq1_ref[...]) + bq1_ref[...])
        q = _mmT(q, wq2_ref[...]) + bq2_ref[...]
        K = q.shape[-1]

        diff3 = s[:, None, :] - bk[None, :, :]
        dist3 = jnp.sqrt(jnp.sum(diff3 * diff3, axis=-1, keepdims=True))
        diff2 = diff3.reshape(Bsz * N, -1)
        dist2 = dist3.reshape(Bsz * N, 1)
        bank_term = _mmT(bk, wr1b_ref[...]) + br1_ref[...]
        pair = (jnp.broadcast_to(bank_term[None],
                                 (Bsz, N, bank_term.shape[-1]))
                .reshape(Bsz * N, -1)
                + _mmT(diff2, wr1d_ref[...])
                + dist2 * wr1dist_ref[...])
        k2 = _mmT(_gelu(pair), wr2_ref[...]) + br2_ref[...]
        k3 = k2.reshape(Bsz, N, K)
        logits = jnp.sum(k3 * q[:, None, :], axis=-1) * (1.0 / math.sqrt(K))
        attn = jax.nn.softmax(logits, axis=-1)
        v = _mmT(bk, wv_ref[...]) + bv_ref[...]
        z = jnp.dot(attn, v)
        rho = _gelu(_mmT(q, wrho1q_ref[...]) + _mmT(z, wrho1z_ref[...])
                    + brho1_ref[...])
        alpha = jax.nn.sigmoid(jnp.dot(rho, wrho2_ref[...],
                                       preferred_element_type=jnp.float32)
                               + brho2_ref[...])
        alpha_ref[...] = jnp.broadcast_to(alpha, alpha_ref.shape)

    wc1_blk = wc1_ref[...]
    wc1_t = wc1_blk[:, D:2 * D]
    wc1_s = wc1_blk[:, 2 * D:]
    s_proj = _mmT(s_ref[...], ws_ref[...]) + bs_ref[...]
    bias1_ref[...] = (_mmT(s_proj, wc1_s) + bc1_ref[...]
                      + _mmT(bt_ref[...], wc1_t))
    wtfT_ref[...] = jnp.dot(wc1_t, wt_ref[...],
                            preferred_element_type=jnp.float32
                            ).astype(jnp.bfloat16)
    w1b_ref[...] = wc1_blk[:, :D].astype(jnp.bfloat16)
    w2b_ref[...] = wc2_ref[...].astype(jnp.bfloat16)


def _mlp_kernel(x_ref, t_ref, alpha_ref, bias1_ref,
                w1_ref, wtfT_ref, w2_ref, bc2_ref, lng_ref, lnb_ref,
                out_ref):
    sub = BLK_T // SPLIT
    D = x_ref.shape[-1]
    bias1 = bias1_ref[0]
    alpha = alpha_ref[0, 0, 0]
    aln = alpha * lng_ref[...]
    alnb = alpha * lnb_ref[...]
    inv_d = 1.0 / D
    for i in range(SPLIT):
        sl = pl.ds(i * sub, sub)
        x = x_ref[0, sl, :]
        h1 = _mmT(x.astype(jnp.bfloat16), w1_ref[...])
        h1 = h1 + _mmT(t_ref[0, sl, :].astype(jnp.bfloat16), wtfT_ref[...])
        h1 = h1 + bias1
        g = _gelu(h1).astype(jnp.bfloat16)
        corr = _mmT(g, w2_ref[...]) + bc2_ref[...]
        mu = jnp.sum(corr, axis=-1, keepdims=True) * inv_d
        m2 = jnp.sum(corr * corr, axis=-1, keepdims=True) * inv_d
        var = m2 - mu * mu
        r = jax.lax.rsqrt(var + 1e-5)
        out_ref[0, sl, :] = x + ((corr - mu) * r) * aln + alnb


def kernel(hidden_states, static_features, time_features, bank,
           Wq1, bq1, Wq2, bq2, Wr1, br1, Wr2, br2, Wv, bv,
           Wrho1, brho1, Wrho2, brho2, Wt, bt, Ws, bs,
           Wc1, bc1, Wc2, bc2, ln_g, ln_b):
    f32 = jnp.float32
    bf16 = jnp.bfloat16
    B, T, D = hidden_states.shape
    NS = static_features.shape[1]
    NT = time_features.shape[2]
    N = bank.shape[0]
    K = Wq2.shape[0]
    DFF = Wc1.shape[0]
    GB = DFF // GATE_G

    row = lambda b: b.reshape(1, -1)
    full = lambda *shape: pl.BlockSpec(shape, lambda g: (0,) * len(shape))

    alpha8, bias1, wtfT, w1b, w2b = pl.pallas_call(
        _gate_kernel,
        grid=(GATE_G,),
        in_specs=[
            full(B, NS), full(N, NS),
            full(2 * K, NS), full(1, 2 * K), full(K, 2 * K), full(1, K),
            full(2 * K, NS), full(2 * K, NS), full(1, 2 * K), full(1, 2 * K),
            full(K, 2 * K), full(1, K), full(K, NS), full(1, K),
            full(K, K), full(K, K), full(1, K), full(K, 1), full(1, 1),
            full(D, NS), full(1, D),
            pl.BlockSpec((GB, 3 * D), lambda g: (g, 0)),
            pl.BlockSpec((D, GB), lambda g: (0, g)),
            full(D, NT), full(1, D),
            pl.BlockSpec((1, GB), lambda g: (0, g)),
        ],
        out_specs=[
            pl.BlockSpec((B, 128), lambda g: (0, 0)),
            pl.BlockSpec((B, GB), lambda g: (0, g)),
            pl.BlockSpec((GB, NT), lambda g: (g, 0)),
            pl.BlockSpec((GB, D), lambda g: (g, 0)),
            pl.BlockSpec((D, GB), lambda g: (0, g)),
        ],
        out_shape=[
            jax.ShapeDtypeStruct((B, 128), f32),
            jax.ShapeDtypeStruct((B, DFF), f32),
            jax.ShapeDtypeStruct((DFF, NT), bf16),
            jax.ShapeDtypeStruct((DFF, D), bf16),
            jax.ShapeDtypeStruct((D, DFF), bf16),
        ],
    )(static_features, bank,
      Wq1, row(bq1), Wq2, row(bq2),
      Wr1[:, :NS], Wr1[:, NS:2 * NS], Wr1[:, 2 * NS:].T, row(br1),
      Wr2, row(br2), Wv, row(bv),
      Wrho1[:, :K], Wrho1[:, K:], row(brho1), Wrho2.T, row(brho2),
      Ws, row(bs), Wc1, Wc2, Wt, row(bt), row(bc1))

    n_tblk = T // BLK_T
    out = pl.pallas_call(
        _mlp_kernel,
        grid=(B, n_tblk),
        in_specs=[
            pl.BlockSpec((1, BLK_T, D), lambda b, t: (b, t, 0)),
            pl.BlockSpec((1, BLK_T, NT), lambda b, t: (b, t, 0)),
            pl.BlockSpec((1, 1, 128), lambda b, t: (b, 0, 0)),
            pl.BlockSpec((1, 1, DFF), lambda b, t: (b, 0, 0)),
            pl.BlockSpec((DFF, D), lambda b, t: (0, 0)),
            pl.BlockSpec((DFF, NT), lambda b, t: (0, 0)),
            pl.BlockSpec((D, DFF), lambda b, t: (0, 0)),
            pl.BlockSpec((1, D), lambda b, t: (0, 0)),
            pl.BlockSpec((1, D), lambda b, t: (0, 0)),
            pl.BlockSpec((1, D), lambda b, t: (0, 0)),
        ],
        out_specs=pl.BlockSpec((1, BLK_T, D), lambda b, t: (b, t, 0)),
        out_shape=jax.ShapeDtypeStruct((B, T, D), f32),
    )(hidden_states, time_features,
      alpha8.reshape(B, 1, 128), bias1.reshape(B, 1, DFF),
      w1b, wtfT, w2b,
      row(bc2), row(ln_g), row(ln_b))
    return out
